# baseline (speedup 1.0000x reference)
import math

import jax
import jax.numpy as jnp
import numpy as np
from jax.experimental import pallas as pl
from jax.experimental.pallas import tpu as pltpu

H = 16
B = 2
T = 2048
C = 1024
D = C // H
U = 40
BT = 256
TT = T // BT
HP = 2
_HIGHEST = jax.lax.Precision.HIGHEST


def _rotl32(x, d):
    d = np.uint32(d)
    return (x << d) | (x >> np.uint32(32 - d))


def _threefry2x32(k1, k2, x1, x2):
    rot = ([13, 15, 26, 6], [17, 29, 16, 24])
    ks = [np.uint32(k1), np.uint32(k2),
          np.uint32(k1) ^ np.uint32(k2) ^ np.uint32(0x1BD11BDA)]
    x = [x1.astype(np.uint32) + ks[0], x2.astype(np.uint32) + ks[1]]

    def rnds(x, rots):
        for r in rots:
            x[0] = x[0] + x[1]
            x[1] = x[0] ^ _rotl32(x[1], r)
        return x

    x = rnds(x, rot[0]); x[0] += ks[1]; x[1] += ks[2] + np.uint32(1)
    x = rnds(x, rot[1]); x[0] += ks[2]; x[1] += ks[0] + np.uint32(2)
    x = rnds(x, rot[0]); x[0] += ks[0]; x[1] += ks[1] + np.uint32(3)
    x = rnds(x, rot[1]); x[0] += ks[1]; x[1] += ks[2] + np.uint32(4)
    x = rnds(x, rot[0]); x[0] += ks[2]; x[1] += ks[0] + np.uint32(5)
    return x[0], x[1]


def _build_sample_consts():
    b1, b2 = _threefry2x32(np.uint32(0), np.uint32(42),
                           np.zeros(2, np.uint32), np.arange(2, dtype=np.uint32))
    i = np.arange(T * U, dtype=np.uint64)
    c1 = (i >> np.uint64(32)).astype(np.uint32)
    c2 = (i & np.uint64(0xFFFFFFFF)).astype(np.uint32)
    o1, o2 = _threefry2x32(b1[1], b2[1], c1, c2)
    idx_np = ((o1 ^ o2) % np.uint32(T)).astype(np.int64).reshape(T, U)
    cnt = np.zeros((T, T), dtype=np.float32)
    np.add.at(cnt, (np.arange(T)[:, None], idx_np), 1.0)
    maskadd = np.where(cnt > 0, 0.0, -np.inf).astype(np.float32)
    return np.ascontiguousarray(cnt.T), np.ascontiguousarray(maskadd.T)


_CNT_T, _MASKADD_T = _build_sample_consts()
_TRI = np.tril(np.ones((BT, BT), dtype=np.float32))


def _proj_kernel(x_ref, wq_ref, wk_ref, wv_ref, bq_ref, bk_ref, bv_ref,
                 q_ref, k_ref, v_ref, vbtc_ref):
    xt = x_ref[0]
    qt = jnp.dot(xt, wq_ref[...], preferred_element_type=jnp.float32) + bq_ref[...]
    kt = jnp.dot(xt, wk_ref[...], preferred_element_type=jnp.float32) + bk_ref[...]
    vt = jnp.dot(xt, wv_ref[...], preferred_element_type=jnp.float32) + bv_ref[...]
    vbtc_ref[0] = vt
    for hh in range(H):
        q_ref[0, hh] = qt[:, hh * D:(hh + 1) * D]
        k_ref[0, hh] = kt[:, hh * D:(hh + 1) * D]
        v_ref[0, hh] = vt[:, hh * D:(hh + 1) * D]


def _m_kernel(q_ref, k_ref, cnt_ref, mask_ref, m_ref):
    for hh in range(HP):
        k_all = k_ref[0, hh]
        for tq in range(TT):
            qt = q_ref[0, hh, tq * BT:(tq + 1) * BT, :]
            sT = jax.lax.dot_general(k_all, qt, (((1,), (1,)), ((), ())),
                                     preferred_element_type=jnp.float32)
            mx = jnp.max(sT + mask_ref[:, tq * BT:(tq + 1) * BT],
                         axis=0, keepdims=True)
            sm = jnp.sum(sT * cnt_ref[:, tq * BT:(tq + 1) * BT],
                         axis=0, keepdims=True) * (1.0 / T)
            m_ref[0, hh, tq:tq + 1, :] = mx - sm


def _topk_kernel(m_ref, idx_ref):
    BH = B * H
    lanes = jax.lax.broadcasted_iota(jnp.int32, (BH, T), 1)
    ulanes = jax.lax.broadcasted_iota(jnp.int32, (BH, U), 1)

    def body(i, carry):
        m, idxmat = carry
        rowmax = jnp.max(m, axis=1, keepdims=True)
        idx = jnp.min(jnp.where(m == rowmax, lanes, 2 * T),
                      axis=1, keepdims=True)
        idxmat = jnp.where(ulanes == i, idx, idxmat)
        m = jnp.where(lanes == idx, -jnp.inf, m)
        return m, idxmat

    _, idxmat = jax.lax.fori_loop(
        0, U, body, (m_ref[...], jnp.zeros((BH, U), jnp.int32)))
    idx_ref[...] = idxmat


def _attn_kernel(idx_sref, q_ref, k_ref, v_ref, updfull_ref, midx_s, dsel_s):
    b = pl.program_id(0)
    hp = pl.program_id(1)
    col = jax.lax.broadcasted_iota(jnp.int32, (U, T), 1).astype(jnp.float32)
    for hh in range(HP):
        base = (b * H + hp * HP + hh) * U
        for i in range(U):
            ti = idx_sref[base + i]
            midx_s[i:i + 1, :] = jnp.reshape(ti.astype(jnp.float32), (1, 1))
            dsel_s[i:i + 1, :] = q_ref[0, hh, pl.ds(ti, 1), :]
        midx = midx_s[...]
        qsel = dsel_s[...]

        k_all = k_ref[0, hh]
        v_all = v_ref[0, hh]
        causal_b = col <= midx
        causal = causal_b.astype(jnp.float32)

        scores = jax.lax.dot_general(qsel, k_all, (((1,), (1,)), ((), ())),
                                     preferred_element_type=jnp.float32)
        scores = scores * (1.0 / math.sqrt(D))
        scores = jnp.where(causal_b, scores, -jnp.inf)
        scores = scores - jnp.max(scores, axis=1, keepdims=True)
        p = jnp.exp(scores)
        attn = p / jnp.sum(p, axis=1, keepdims=True)
        upd = jax.lax.dot_general(attn, v_all, (((1,), (0,)), ((), ())),
                                  preferred_element_type=jnp.float32)
        cumct = jnp.dot(causal, v_all, preferred_element_type=jnp.float32,
                        precision=_HIGHEST)
        dsel_s[...] = upd - cumct
        updfull_ref[0, hh] = jnp.zeros((T, D), jnp.float32)
        for i in range(U):
            ti = idx_sref[base + i]
            updfull_ref[0, hh, pl.ds(ti, 1), :] = dsel_s[i:i + 1, :]


def _ctx_kernel(vbtc_ref, updfull_ref, tri_ref, wp_ref, bp_ref,
                out_ref, carry_s):
    t = pl.program_id(1)

    @pl.when(t == 0)
    def _init_carry():
        carry_s[...] = jnp.zeros((1, C), jnp.float32)

    vt = vbtc_ref[0]
    ctx = jnp.dot(tri_ref[...], vt, preferred_element_type=jnp.float32,
                  precision=_HIGHEST)
    ctx = ctx + carry_s[...]
    carry_s[...] = ctx[BT - 1:BT, :]
    updtile = jnp.concatenate([updfull_ref[0, hh] for hh in range(H)], axis=1)
    y = ctx + updtile
    out_ref[0] = (jnp.dot(y, wp_ref[...], preferred_element_type=jnp.float32)
                  + bp_ref[...])


def kernel(x, Wq, bq, Wk, bk, Wv, bv, Wp, bp):
    bq2 = bq.reshape(1, C)
    bk2 = bk.reshape(1, C)
    bv2 = bv.reshape(1, C)
    bp2 = bp.reshape(1, C)
    cntT = jnp.asarray(_CNT_T)
    maskT = jnp.asarray(_MASKADD_T)
    tri = jnp.asarray(_TRI)
    f32 = jnp.float32

    q, k, v, vbtc = pl.pallas_call(
        _proj_kernel,
        grid=(B, TT),
        in_specs=[
            pl.BlockSpec((1, BT, C), lambda b, t: (b, t, 0)),
            pl.BlockSpec((C, C), lambda b, t: (0, 0)),
            pl.BlockSpec((C, C), lambda b, t: (0, 0)),
            pl.BlockSpec((C, C), lambda b, t: (0, 0)),
            pl.BlockSpec((1, C), lambda b, t: (0, 0)),
            pl.BlockSpec((1, C), lambda b, t: (0, 0)),
            pl.BlockSpec((1, C), lambda b, t: (0, 0)),
        ],
        out_specs=[
            pl.BlockSpec((1, H, BT, D), lambda b, t: (b, 0, t, 0)),
            pl.BlockSpec((1, H, BT, D), lambda b, t: (b, 0, t, 0)),
            pl.BlockSpec((1, H, BT, D), lambda b, t: (b, 0, t, 0)),
            pl.BlockSpec((1, BT, C), lambda b, t: (b, t, 0)),
        ],
        out_shape=[
            jax.ShapeDtypeStruct((B, H, T, D), f32),
            jax.ShapeDtypeStruct((B, H, T, D), f32),
            jax.ShapeDtypeStruct((B, H, T, D), f32),
            jax.ShapeDtypeStruct((B, T, C), f32),
        ],
        compiler_params=pltpu.CompilerParams(
            dimension_semantics=("parallel", "parallel")),
    )(x, Wq, Wk, Wv, bq2, bk2, bv2)

    m = pl.pallas_call(
        _m_kernel,
        grid=(B, H // HP),
        in_specs=[
            pl.BlockSpec((1, HP, T, D), lambda b, h: (b, h, 0, 0)),
            pl.BlockSpec((1, HP, T, D), lambda b, h: (b, h, 0, 0)),
            pl.BlockSpec((T, T), lambda b, h: (0, 0)),
            pl.BlockSpec((T, T), lambda b, h: (0, 0)),
        ],
        out_specs=pl.BlockSpec((1, HP, TT, BT), lambda b, h: (b, h, 0, 0)),
        out_shape=jax.ShapeDtypeStruct((B, H, TT, BT), f32),
        compiler_params=pltpu.CompilerParams(
            dimension_semantics=("parallel", "parallel")),
    )(q, k, cntT, maskT)

    idxmat = pl.pallas_call(
        _topk_kernel,
        grid=(1,),
        in_specs=[pl.BlockSpec((B * H, T), lambda i: (0, 0))],
        out_specs=pl.BlockSpec((B * H, U), lambda i: (0, 0)),
        out_shape=jax.ShapeDtypeStruct((B * H, U), jnp.int32),
    )(m.reshape(B * H, T))

    updfull = pl.pallas_call(
        _attn_kernel,
        grid_spec=pltpu.PrefetchScalarGridSpec(
            num_scalar_prefetch=1,
            grid=(B, H // HP),
            in_specs=[
                pl.BlockSpec((1, HP, T, D), lambda b, h, *_: (b, h, 0, 0)),
                pl.BlockSpec((1, HP, T, D), lambda b, h, *_: (b, h, 0, 0)),
                pl.BlockSpec((1, HP, T, D), lambda b, h, *_: (b, h, 0, 0)),
            ],
            out_specs=pl.BlockSpec((1, HP, T, D), lambda b, h, *_: (b, h, 0, 0)),
            scratch_shapes=[pltpu.VMEM((U, 1), f32), pltpu.VMEM((U, D), f32)],
        ),
        out_shape=jax.ShapeDtypeStruct((B, H, T, D), f32),
        compiler_params=pltpu.CompilerParams(
            dimension_semantics=("parallel", "parallel")),
    )(idxmat.reshape(B * H * U), q, k, v)

    out = pl.pallas_call(
        _ctx_kernel,
        grid=(B, TT),
        in_specs=[
            pl.BlockSpec((1, BT, C), lambda b, t: (b, t, 0)),
            pl.BlockSpec((1, H, BT, D), lambda b, t: (b, 0, t, 0)),
            pl.BlockSpec((BT, BT), lambda b, t: (0, 0)),
            pl.BlockSpec((C, C), lambda b, t: (0, 0)),
            pl.BlockSpec((1, C), lambda b, t: (0, 0)),
        ],
        out_specs=pl.BlockSpec((1, BT, C), lambda b, t: (b, t, 0)),
        out_shape=jax.ShapeDtypeStruct((B, T, C), f32),
        scratch_shapes=[pltpu.VMEM((1, C), f32)],
        compiler_params=pltpu.CompilerParams(
            dimension_semantics=("parallel", "arbitrary")),
    )(vbtc, updfull, tri, Wp, bp2)

    return out


# speedup vs baseline: 1.0151x; 1.0151x over previous
import math

import jax
import jax.numpy as jnp
import numpy as np
from jax.experimental import pallas as pl
from jax.experimental.pallas import tpu as pltpu

H = 16
B = 2
T = 2048
C = 1024
D = C // H
U = 40
BT = 256
TT = T // BT
HP = 2
_HIGHEST = jax.lax.Precision.HIGHEST


def _rotl32(x, d):
    d = np.uint32(d)
    return (x << d) | (x >> np.uint32(32 - d))


def _threefry2x32(k1, k2, x1, x2):
    rot = ([13, 15, 26, 6], [17, 29, 16, 24])
    ks = [np.uint32(k1), np.uint32(k2),
          np.uint32(k1) ^ np.uint32(k2) ^ np.uint32(0x1BD11BDA)]
    x = [x1.astype(np.uint32) + ks[0], x2.astype(np.uint32) + ks[1]]

    def rnds(x, rots):
        for r in rots:
            x[0] = x[0] + x[1]
            x[1] = x[0] ^ _rotl32(x[1], r)
        return x

    x = rnds(x, rot[0]); x[0] += ks[1]; x[1] += ks[2] + np.uint32(1)
    x = rnds(x, rot[1]); x[0] += ks[2]; x[1] += ks[0] + np.uint32(2)
    x = rnds(x, rot[0]); x[0] += ks[0]; x[1] += ks[1] + np.uint32(3)
    x = rnds(x, rot[1]); x[0] += ks[1]; x[1] += ks[2] + np.uint32(4)
    x = rnds(x, rot[0]); x[0] += ks[2]; x[1] += ks[0] + np.uint32(5)
    return x[0], x[1]


def _build_sample_consts():
    b1, b2 = _threefry2x32(np.uint32(0), np.uint32(42),
                           np.zeros(2, np.uint32), np.arange(2, dtype=np.uint32))
    i = np.arange(T * U, dtype=np.uint64)
    c1 = (i >> np.uint64(32)).astype(np.uint32)
    c2 = (i & np.uint64(0xFFFFFFFF)).astype(np.uint32)
    o1, o2 = _threefry2x32(b1[1], b2[1], c1, c2)
    idx_np = ((o1 ^ o2) % np.uint32(T)).astype(np.int64).reshape(T, U)
    cnt = np.zeros((T, T), dtype=np.float32)
    np.add.at(cnt, (np.arange(T)[:, None], idx_np), 1.0)
    maskadd = np.where(cnt > 0, 0.0, -np.inf).astype(np.float32)
    return np.ascontiguousarray(cnt.T), np.ascontiguousarray(maskadd.T)


_CNT_T, _MASKADD_T = _build_sample_consts()
_TRI = np.tril(np.ones((BT, BT), dtype=np.float32))


def _proj_kernel(x_ref, wq_ref, wk_ref, wv_ref, bq_ref, bk_ref, bv_ref,
                 q_ref, k_ref, v_ref, vbtc_ref):
    xt = x_ref[0]
    qt = jnp.dot(xt, wq_ref[...], preferred_element_type=jnp.float32) + bq_ref[...]
    kt = jnp.dot(xt, wk_ref[...], preferred_element_type=jnp.float32) + bk_ref[...]
    vt = jnp.dot(xt, wv_ref[...], preferred_element_type=jnp.float32) + bv_ref[...]
    vbtc_ref[0] = vt
    for hh in range(H):
        q_ref[0, hh] = qt[:, hh * D:(hh + 1) * D]
        k_ref[0, hh] = kt[:, hh * D:(hh + 1) * D]
        v_ref[0, hh] = vt[:, hh * D:(hh + 1) * D]


def _m_topk_kernel(q_ref, k_ref, cnt_ref, mask_ref, idx_ref, m_s):
    b = pl.program_id(0)
    hp = pl.program_id(1)
    for hh in range(HP):
        k_all = k_ref[0, hh]
        bh = b * H + hp * HP + hh
        for tq in range(TT):
            qt = q_ref[0, hh, tq * BT:(tq + 1) * BT, :]
            sT = jax.lax.dot_general(k_all, qt, (((1,), (1,)), ((), ())),
                                     preferred_element_type=jnp.float32)
            mx = jnp.max(sT + mask_ref[:, tq * BT:(tq + 1) * BT],
                         axis=0, keepdims=True)
            sm = jnp.sum(sT * cnt_ref[:, tq * BT:(tq + 1) * BT],
                         axis=0, keepdims=True) * (1.0 / T)
            m_s[pl.ds(bh, 1), tq * BT:(tq + 1) * BT] = mx - sm

    @pl.when(jnp.logical_and(b == B - 1, hp == H // HP - 1))
    def _topk():
        BH = B * H
        lanes = jax.lax.broadcasted_iota(jnp.int32, (BH, T), 1)
        ulanes = jax.lax.broadcasted_iota(jnp.int32, (BH, U), 1)

        def body(i, carry):
            m, idxmat = carry
            rowmax = jnp.max(m, axis=1, keepdims=True)
            idx = jnp.min(jnp.where(m == rowmax, lanes, 2 * T),
                          axis=1, keepdims=True)
            idxmat = jnp.where(ulanes == i, idx, idxmat)
            m = jnp.where(lanes == idx, -jnp.inf, m)
            return m, idxmat

        _, idxmat = jax.lax.fori_loop(
            0, U, body, (m_s[...], jnp.zeros((BH, U), jnp.int32)))
        idx_ref[...] = idxmat


def _attn_kernel(idx_sref, q_ref, k_ref, v_ref, updfull_ref, midx_s, dsel_s):
    b = pl.program_id(0)
    hp = pl.program_id(1)
    col = jax.lax.broadcasted_iota(jnp.int32, (U, T), 1).astype(jnp.float32)
    for hh in range(HP):
        base = (b * H + hp * HP + hh) * U
        for i in range(U):
            ti = idx_sref[base + i]
            midx_s[i:i + 1, :] = jnp.reshape(ti.astype(jnp.float32), (1, 1))
            dsel_s[i:i + 1, :] = q_ref[0, hh, pl.ds(ti, 1), :]
        midx = midx_s[...]
        qsel = dsel_s[...]

        k_all = k_ref[0, hh]
        v_all = v_ref[0, hh]
        causal_b = col <= midx
        causal = causal_b.astype(jnp.float32)

        scores = jax.lax.dot_general(qsel, k_all, (((1,), (1,)), ((), ())),
                                     preferred_element_type=jnp.float32)
        scores = scores * (1.0 / math.sqrt(D))
        scores = jnp.where(causal_b, scores, -jnp.inf)
        scores = scores - jnp.max(scores, axis=1, keepdims=True)
        p = jnp.exp(scores)
        attn = p / jnp.sum(p, axis=1, keepdims=True)
        upd = jax.lax.dot_general(attn, v_all, (((1,), (0,)), ((), ())),
                                  preferred_element_type=jnp.float32)
        cumct = jnp.dot(causal, v_all, preferred_element_type=jnp.float32,
                        precision=_HIGHEST)
        dsel_s[...] = upd - cumct
        updfull_ref[0, hh] = jnp.zeros((T, D), jnp.float32)
        for i in range(U):
            ti = idx_sref[base + i]
            updfull_ref[0, hh, pl.ds(ti, 1), :] = dsel_s[i:i + 1, :]


def _ctx_kernel(vbtc_ref, updfull_ref, tri_ref, wp_ref, bp_ref,
                out_ref, carry_s):
    t = pl.program_id(1)

    @pl.when(t == 0)
    def _init_carry():
        carry_s[...] = jnp.zeros((1, C), jnp.float32)

    vt = vbtc_ref[0]
    ctx = jnp.dot(tri_ref[...], vt, preferred_element_type=jnp.float32,
                  precision=_HIGHEST)
    ctx = ctx + carry_s[...]
    carry_s[...] = ctx[BT - 1:BT, :]
    updtile = jnp.concatenate([updfull_ref[0, hh] for hh in range(H)], axis=1)
    y = ctx + updtile
    out_ref[0] = (jnp.dot(y, wp_ref[...], preferred_element_type=jnp.float32)
                  + bp_ref[...])


def kernel(x, Wq, bq, Wk, bk, Wv, bv, Wp, bp):
    bq2 = bq.reshape(1, C)
    bk2 = bk.reshape(1, C)
    bv2 = bv.reshape(1, C)
    bp2 = bp.reshape(1, C)
    cntT = jnp.asarray(_CNT_T)
    maskT = jnp.asarray(_MASKADD_T)
    tri = jnp.asarray(_TRI)
    f32 = jnp.float32

    q, k, v, vbtc = pl.pallas_call(
        _proj_kernel,
        grid=(B, TT),
        in_specs=[
            pl.BlockSpec((1, BT, C), lambda b, t: (b, t, 0)),
            pl.BlockSpec((C, C), lambda b, t: (0, 0)),
            pl.BlockSpec((C, C), lambda b, t: (0, 0)),
            pl.BlockSpec((C, C), lambda b, t: (0, 0)),
            pl.BlockSpec((1, C), lambda b, t: (0, 0)),
            pl.BlockSpec((1, C), lambda b, t: (0, 0)),
            pl.BlockSpec((1, C), lambda b, t: (0, 0)),
        ],
        out_specs=[
            pl.BlockSpec((1, H, BT, D), lambda b, t: (b, 0, t, 0)),
            pl.BlockSpec((1, H, BT, D), lambda b, t: (b, 0, t, 0)),
            pl.BlockSpec((1, H, BT, D), lambda b, t: (b, 0, t, 0)),
            pl.BlockSpec((1, BT, C), lambda b, t: (b, t, 0)),
        ],
        out_shape=[
            jax.ShapeDtypeStruct((B, H, T, D), f32),
            jax.ShapeDtypeStruct((B, H, T, D), f32),
            jax.ShapeDtypeStruct((B, H, T, D), f32),
            jax.ShapeDtypeStruct((B, T, C), f32),
        ],
        compiler_params=pltpu.CompilerParams(
            dimension_semantics=("parallel", "parallel")),
    )(x, Wq, Wk, Wv, bq2, bk2, bv2)

    idxmat = pl.pallas_call(
        _m_topk_kernel,
        grid=(B, H // HP),
        in_specs=[
            pl.BlockSpec((1, HP, T, D), lambda b, h: (b, h, 0, 0)),
            pl.BlockSpec((1, HP, T, D), lambda b, h: (b, h, 0, 0)),
            pl.BlockSpec((T, T), lambda b, h: (0, 0)),
            pl.BlockSpec((T, T), lambda b, h: (0, 0)),
        ],
        out_specs=pl.BlockSpec((B * H, U), lambda b, h: (0, 0)),
        out_shape=jax.ShapeDtypeStruct((B * H, U), jnp.int32),
        scratch_shapes=[pltpu.VMEM((B * H, T), f32)],
        compiler_params=pltpu.CompilerParams(
            dimension_semantics=("arbitrary", "arbitrary")),
    )(q, k, cntT, maskT)

    updfull = pl.pallas_call(
        _attn_kernel,
        grid_spec=pltpu.PrefetchScalarGridSpec(
            num_scalar_prefetch=1,
            grid=(B, H // HP),
            in_specs=[
                pl.BlockSpec((1, HP, T, D), lambda b, h, *_: (b, h, 0, 0)),
                pl.BlockSpec((1, HP, T, D), lambda b, h, *_: (b, h, 0, 0)),
                pl.BlockSpec((1, HP, T, D), lambda b, h, *_: (b, h, 0, 0)),
            ],
            out_specs=pl.BlockSpec((1, HP, T, D), lambda b, h, *_: (b, h, 0, 0)),
            scratch_shapes=[pltpu.VMEM((U, 1), f32), pltpu.VMEM((U, D), f32)],
        ),
        out_shape=jax.ShapeDtypeStruct((B, H, T, D), f32),
        compiler_params=pltpu.CompilerParams(
            dimension_semantics=("parallel", "parallel")),
    )(idxmat.reshape(B * H * U), q, k, v)

    out = pl.pallas_call(
        _ctx_kernel,
        grid=(B, TT),
        in_specs=[
            pl.BlockSpec((1, BT, C), lambda b, t: (b, t, 0)),
            pl.BlockSpec((1, H, BT, D), lambda b, t: (b, 0, t, 0)),
            pl.BlockSpec((BT, BT), lambda b, t: (0, 0)),
            pl.BlockSpec((C, C), lambda b, t: (0, 0)),
            pl.BlockSpec((1, C), lambda b, t: (0, 0)),
        ],
        out_specs=pl.BlockSpec((1, BT, C), lambda b, t: (b, t, 0)),
        out_shape=jax.ShapeDtypeStruct((B, T, C), f32),
        scratch_shapes=[pltpu.VMEM((1, C), f32)],
        compiler_params=pltpu.CompilerParams(
            dimension_semantics=("parallel", "arbitrary")),
    )(vbtc, updfull, tri, Wp, bp2)

    return out
